# Initial kernel scaffold; baseline (speedup 1.0000x reference)
import jax
import jax.numpy as jnp
from jax.experimental import pallas as pl


def kernel(feature_low, coord_low, feature_high, coord_high, W_low, b_low, gamma_low, beta_low, W_high, b_high, gamma_high, beta_high):
    raise NotImplementedError("write your pallas kernel here")


# speedup vs baseline: 25.8493x; 25.8493x over previous
import functools

import jax
import jax.numpy as jnp
from jax.experimental import pallas as pl

_HI = jax.lax.Precision.HIGHEST


def _mm_stats_kernel(x_ref, w_ref, b_ref, y_ref, s_ref):
    y = jax.lax.dot_general(x_ref[...], w_ref[...], (((1,), (1,)), ((), ())),
                            preferred_element_type=jnp.float32)
    y = y + b_ref[...]
    y_ref[...] = y

    @pl.when(pl.program_id(0) == 0)
    def _init():
        s_ref[...] = jnp.zeros_like(s_ref)

    s = jnp.concatenate([jnp.sum(y, axis=0, keepdims=True),
                         jnp.sum(y * y, axis=0, keepdims=True)], axis=0)
    s_ref[...] += s


def _mm_stats(x2d, w, b, blk):
    n, d_in = x2d.shape
    d_out = w.shape[0]
    return pl.pallas_call(
        _mm_stats_kernel,
        grid=(n // blk,),
        in_specs=[
            pl.BlockSpec((blk, d_in), lambda i: (i, 0)),
            pl.BlockSpec((d_out, d_in), lambda i: (0, 0)),
            pl.BlockSpec((1, d_out), lambda i: (0, 0)),
        ],
        out_specs=[
            pl.BlockSpec((blk, d_out), lambda i: (i, 0)),
            pl.BlockSpec((2, d_out), lambda i: (0, 0)),
        ],
        out_shape=[
            jax.ShapeDtypeStruct((n, d_out), jnp.float32),
            jax.ShapeDtypeStruct((2, d_out), jnp.float32),
        ],
    )(x2d, w, b.reshape(1, d_out))


def _bn_scale_shift(s, n, gamma, beta):
    mean = s[0] / n
    var = s[1] / n - mean * mean
    scale = gamma * jax.lax.rsqrt(var + 1e-5)
    shift = beta - mean * scale
    return scale.reshape(1, -1), shift.reshape(1, -1)


def _interp_kernel(ch_ref, clt_ref, nh_ref, nl_ref, yl_ref, yh_ref,
                   ssl_ref, ssh_ref, o_ref, *, n_low):
    ch = ch_ref[0]
    clt = clt_ref[0]
    fl = jnp.maximum(yl_ref[0] * ssl_ref[0:1] + ssl_ref[1:2], 0.0)
    fh = jnp.maximum(yh_ref[0] * ssh_ref[0:1] + ssh_ref[1:2], 0.0)

    dot = jax.lax.dot_general(ch, clt, (((1,), (0,)), ((), ())),
                              preferred_element_type=jnp.float32)
    dist = -2.0 * dot
    dist = dist + nh_ref[0]
    dist = dist + nl_ref[0]

    iota = jax.lax.broadcasted_iota(jnp.int32, dist.shape, 1)
    d = dist
    recips = []
    onehots = []
    for _ in range(3):
        m = jnp.min(d, axis=1, keepdims=True)
        am = jnp.min(jnp.where(d == m, iota, n_low), axis=1, keepdims=True)
        sel = iota == am
        recips.append(1.0 / (m + 1e-8))
        onehots.append(sel)
        d = jnp.where(sel, jnp.inf, d)
    wsum = recips[0] + recips[1] + recips[2]
    w_oh = jnp.zeros_like(dist)
    for k in range(3):
        w_oh = w_oh + jnp.where(onehots[k], recips[k] / wsum, 0.0)

    interp = jax.lax.dot_general(w_oh, fl, (((1,), (0,)), ((), ())),
                                 precision=_HI,
                                 preferred_element_type=jnp.float32)
    o_ref[0] = interp + fh


def kernel(feature_low, coord_low, feature_high, coord_high,
           W_low, b_low, gamma_low, beta_low,
           W_high, b_high, gamma_high, beta_high):
    B, n_low, d_in_low = feature_low.shape
    _, n_high, d_in_high = feature_high.shape
    d_out = W_low.shape[0]

    yl, sl = _mm_stats(feature_low.reshape(B * n_low, d_in_low), W_low, b_low,
                       blk=2048)
    yh, sh = _mm_stats(feature_high.reshape(B * n_high, d_in_high), W_high,
                       b_high, blk=4096)
    scale_l, shift_l = _bn_scale_shift(sl, B * n_low, gamma_low, beta_low)
    scale_h, shift_h = _bn_scale_shift(sh, B * n_high, gamma_high, beta_high)
    ssl = jnp.concatenate([scale_l, shift_l], axis=0)
    ssh = jnp.concatenate([scale_h, shift_h], axis=0)

    yl = yl.reshape(B, n_low, d_out)
    yh = yh.reshape(B, n_high, d_out)
    clt = jnp.transpose(coord_low, (0, 2, 1))
    nh = jnp.sum(coord_high ** 2, axis=-1)[..., None]
    nl = jnp.sum(coord_low ** 2, axis=-1)[:, None, :]

    blk = 256
    grid = (B, n_high // blk)
    out = pl.pallas_call(
        functools.partial(_interp_kernel, n_low=n_low),
        grid=grid,
        in_specs=[
            pl.BlockSpec((1, blk, 3), lambda b, i: (b, i, 0)),
            pl.BlockSpec((1, 3, n_low), lambda b, i: (b, 0, 0)),
            pl.BlockSpec((1, blk, 1), lambda b, i: (b, i, 0)),
            pl.BlockSpec((1, 1, n_low), lambda b, i: (b, 0, 0)),
            pl.BlockSpec((1, n_low, d_out), lambda b, i: (b, 0, 0)),
            pl.BlockSpec((1, blk, d_out), lambda b, i: (b, i, 0)),
            pl.BlockSpec((2, d_out), lambda b, i: (0, 0)),
            pl.BlockSpec((2, d_out), lambda b, i: (0, 0)),
        ],
        out_specs=pl.BlockSpec((1, blk, d_out), lambda b, i: (b, i, 0)),
        out_shape=jax.ShapeDtypeStruct((B, n_high, d_out), jnp.float32),
    )(coord_high, clt, nh, nl, yl, yh, ssl, ssh)

    return (out, coord_high)


# speedup vs baseline: 41.0051x; 1.5863x over previous
import functools

import jax
import jax.numpy as jnp
from jax import lax
from jax.experimental import pallas as pl
from jax.experimental.pallas import tpu as pltpu
from jax.experimental.pallas import tpu_sc as plsc

_NC = 2
_NS = 16
_NW = _NC * _NS
_CH = 64


def _mm_stats_kernel(x_ref, w_ref, b_ref, y_ref, s_ref):
    y = jax.lax.dot_general(x_ref[...], w_ref[...], (((1,), (1,)), ((), ())),
                            preferred_element_type=jnp.float32)
    y = y + b_ref[...]
    y_ref[...] = y

    @pl.when(pl.program_id(0) == 0)
    def _init():
        s_ref[...] = jnp.zeros_like(s_ref)

    s = jnp.concatenate([jnp.sum(y, axis=0, keepdims=True),
                         jnp.sum(y * y, axis=0, keepdims=True)], axis=0)
    s_ref[...] += s


def _mm_stats(x2d, w, b, blk):
    n, d_in = x2d.shape
    d_out = w.shape[0]
    return pl.pallas_call(
        _mm_stats_kernel,
        grid=(n // blk,),
        in_specs=[
            pl.BlockSpec((blk, d_in), lambda i: (i, 0)),
            pl.BlockSpec((d_out, d_in), lambda i: (0, 0)),
            pl.BlockSpec((1, d_out), lambda i: (0, 0)),
        ],
        out_specs=[
            pl.BlockSpec((blk, d_out), lambda i: (i, 0)),
            pl.BlockSpec((2, d_out), lambda i: (0, 0)),
        ],
        out_shape=[
            jax.ShapeDtypeStruct((n, d_out), jnp.float32),
            jax.ShapeDtypeStruct((2, d_out), jnp.float32),
        ],
    )(x2d, w, b.reshape(1, d_out))


def _bn_scale_shift(s, n, gamma, beta):
    mean = s[0] / n
    var = s[1] / n - mean * mean
    scale = gamma * jax.lax.rsqrt(var + 1e-5)
    shift = beta - mean * scale
    return scale.reshape(1, -1), shift.reshape(1, -1)


def _norm_kernel(y_ref, ss_ref, o_ref):
    o_ref[...] = jnp.maximum(y_ref[...] * ss_ref[0:1] + ss_ref[1:2], 0.0)


def _norm_kernel_pad(y_ref, ss_ref, o_ref):
    yn = jnp.maximum(y_ref[...] * ss_ref[0:1] + ss_ref[1:2], 0.0)
    pad = jnp.zeros_like(yn)
    o_ref[...] = jnp.concatenate([yn, pad], axis=1)


def _norm_relu_pad(y2d, ss, blk):
    n, d = y2d.shape
    return pl.pallas_call(
        _norm_kernel_pad,
        grid=(n // blk,),
        in_specs=[
            pl.BlockSpec((blk, d), lambda i: (i, 0)),
            pl.BlockSpec((2, d), lambda i: (0, 0)),
        ],
        out_specs=pl.BlockSpec((blk, 2 * d), lambda i: (i, 0)),
        out_shape=jax.ShapeDtypeStruct((n, 2 * d), jnp.float32),
    )(y2d, ss)


def _norm_relu(y2d, ss, blk):
    n, d = y2d.shape
    return pl.pallas_call(
        _norm_kernel,
        grid=(n // blk,),
        in_specs=[
            pl.BlockSpec((blk, d), lambda i: (i, 0)),
            pl.BlockSpec((2, d), lambda i: (0, 0)),
        ],
        out_specs=pl.BlockSpec((blk, d), lambda i: (i, 0)),
        out_shape=jax.ShapeDtypeStruct((n, d), jnp.float32),
    )(y2d, ss)


def _top3_kernel(ch_ref, clt_ref, nh_ref, nl_ref, idx_ref, w_ref, *, n_low,
                 b_base):
    b = b_base + pl.program_id(0)
    ch = ch_ref[0]
    clt = clt_ref[0]

    dot = jax.lax.dot_general(ch, clt, (((1,), (0,)), ((), ())),
                              preferred_element_type=jnp.float32)
    dist = -2.0 * dot
    dist = dist + nh_ref[0]
    dist = dist + nl_ref[0]

    iota = jax.lax.broadcasted_iota(jnp.int32, dist.shape, 1)
    d = dist
    recips = []
    ams = []
    for r in range(3):
        m = jnp.min(d, axis=1, keepdims=True)
        am = jnp.min(jnp.where(d == m, iota, n_low), axis=1, keepdims=True)
        recips.append(1.0 / (m + 1e-8))
        ams.append(am)
        if r < 2:
            d = jnp.where(iota == am, jnp.inf, d)
    wsum = recips[0] + recips[1] + recips[2]

    idx_ref[0] = jnp.concatenate(ams, axis=1) + b * n_low
    blk = dist.shape[0]
    w_ref[0] = jnp.concatenate(
        [jnp.broadcast_to(r / wsum, (blk, 16)) for r in recips], axis=1)


def _sc_interp(P, D):
    per_w = P // _NW
    n_chunks = per_w // _CH
    mesh = plsc.VectorSubcoreMesh(core_axis_name="c", subcore_axis_name="s")

    @functools.partial(
        pl.kernel, mesh=mesh,
        out_type=jax.ShapeDtypeStruct((P, D), jnp.float32),
        scratch_types=[
            pltpu.VMEM((3 * per_w,), jnp.int32),
            pltpu.VMEM((_CH, 48), jnp.float32),
            pltpu.VMEM((_CH, D), jnp.float32),
            pltpu.VMEM((3 * _CH // 2, 2 * D), jnp.float32),
            pltpu.VMEM((3 * _CH // 2, 2 * D), jnp.float32),
            pltpu.VMEM((_CH, D), jnp.float32),
            pltpu.SemaphoreType.DMA,
        ],
    )
    def k(fl_hbm, fh_hbm, idx_hbm, w48_hbm, out_hbm,
          iv, wv, fhv, r1v, r2v, outv, sem):
        wid = lax.axis_index("s") * _NC + lax.axis_index("c")
        base = wid * per_w
        pltpu.sync_copy(idx_hbm.at[pl.ds(3 * base, 3 * per_w)], iv)

        def chunk_body(c, carry):
            off = base + c * _CH
            ioff = c * 3 * _CH
            cpf = pltpu.async_copy(fh_hbm.at[pl.ds(off, _CH)], fhv, sem)
            cpw = pltpu.async_copy(w48_hbm.at[pl.ds(off, _CH)], wv, sem)
            cp1 = pltpu.async_copy(
                fl_hbm.at[iv.at[pl.ds(ioff, 3 * _CH // 2)]], r1v, sem)
            cp2 = pltpu.async_copy(
                fl_hbm.at[iv.at[pl.ds(ioff + 3 * _CH // 2, 3 * _CH // 2)]],
                r2v, sem)
            cpf.wait()
            cpw.wait()
            cp1.wait()
            cp2.wait()
            for i in range(_CH):
                rv = r1v if i < _CH // 2 else r2v
                j = i if i < _CH // 2 else i - _CH // 2
                w0 = wv[i, pl.ds(0, 16)]
                w1 = wv[i, pl.ds(16, 16)]
                w2 = wv[i, pl.ds(32, 16)]
                for cd in range(D // 16):
                    sl = pl.ds(cd * 16, 16)
                    acc = rv[3 * j, sl] * w0
                    acc = acc + rv[3 * j + 1, sl] * w1
                    acc = acc + rv[3 * j + 2, sl] * w2
                    outv[i, sl] = acc + fhv[i, sl]
            pltpu.sync_copy(outv, out_hbm.at[pl.ds(off, _CH)])
            return carry

        lax.fori_loop(0, n_chunks, chunk_body, 0)

    return k


def kernel(feature_low, coord_low, feature_high, coord_high,
           W_low, b_low, gamma_low, beta_low,
           W_high, b_high, gamma_high, beta_high):
    B, n_low, d_in_low = feature_low.shape
    _, n_high, d_in_high = feature_high.shape
    d_out = W_low.shape[0]

    yl, sl = _mm_stats(feature_low.reshape(B * n_low, d_in_low), W_low, b_low,
                       blk=2048)
    yh, sh = _mm_stats(feature_high.reshape(B * n_high, d_in_high), W_high,
                       b_high, blk=4096)
    scale_l, shift_l = _bn_scale_shift(sl, B * n_low, gamma_low, beta_low)
    scale_h, shift_h = _bn_scale_shift(sh, B * n_high, gamma_high, beta_high)
    ssl = jnp.concatenate([scale_l, shift_l], axis=0)
    ssh = jnp.concatenate([scale_h, shift_h], axis=0)

    fl = _norm_relu_pad(yl, ssl, blk=B * n_low)
    fh = _norm_relu(yh, ssh, blk=8192)

    clt = jnp.transpose(coord_low, (0, 2, 1))
    nh = jnp.sum(coord_high ** 2, axis=-1)[..., None]
    nl = jnp.sum(coord_low ** 2, axis=-1)[:, None, :]

    blk = 1024
    sc_call = _sc_interp(n_high, d_out)
    outs = []
    for b in range(B):
        idx3, w48 = pl.pallas_call(
            functools.partial(_top3_kernel, n_low=n_low, b_base=b),
            grid=(1, n_high // blk),
            in_specs=[
                pl.BlockSpec((1, blk, 3), lambda bb, i: (bb, i, 0)),
                pl.BlockSpec((1, 3, n_low), lambda bb, i: (bb, 0, 0)),
                pl.BlockSpec((1, blk, 1), lambda bb, i: (bb, i, 0)),
                pl.BlockSpec((1, 1, n_low), lambda bb, i: (bb, 0, 0)),
            ],
            out_specs=[
                pl.BlockSpec((1, blk, 3), lambda bb, i: (bb, i, 0)),
                pl.BlockSpec((1, blk, 48), lambda bb, i: (bb, i, 0)),
            ],
            out_shape=[
                jax.ShapeDtypeStruct((1, n_high, 3), jnp.int32),
                jax.ShapeDtypeStruct((1, n_high, 48), jnp.float32),
            ],
        )(coord_high[b:b + 1], clt[b:b + 1], nh[b:b + 1], nl[b:b + 1])
        idxf = idx3.reshape(3 * n_high)
        w48f = w48.reshape(n_high, 48)
        fh_b = jax.lax.slice_in_dim(fh, b * n_high, (b + 1) * n_high, axis=0)
        outs.append(sc_call(fl, fh_b, idxf, w48f))
    out = jnp.stack(outs).reshape(B, n_high, d_out)
    return (out, coord_high)
